# baseline (speedup 1.0000x reference)
import functools

import jax
import jax.numpy as jnp
from jax import lax
from jax.experimental import pallas as pl
from jax.experimental.pallas import tpu as pltpu
from jax.experimental.pallas import tpu_sc as plsc

N = 4096
R = 256
NB = N // R
KNN = 16
NSLOT = 2 * KNN
BIG = 3.0e38

_NC, _NS = 2, 16
_NW = _NC * _NS
_TOTAL_IDX = 4 * NSLOT * N
_PER_W = _TOTAL_IDX // _NW
_CHUNK = 128
_NCHUNK = _PER_W // _CHUNK


CH = 32
POOL = CH * KNN


def _topk_kernel(p1f_ref, p2f_ref, q_ref, oidx_ref, d2_ref,
                 poolv_ref, pooli_ref, done_ref):
    b = pl.program_id(0)
    cs = N // CH
    q = q_ref[0]
    qn = jnp.sum(q * q, axis=0, keepdims=True)
    eye3 = jnp.eye(3, dtype=jnp.float32)
    li3 = lax.broadcasted_iota(jnp.int32, (CH, cs, R), 1)
    chunk_iota = lax.broadcasted_iota(jnp.int32, (CH, R), 0)
    gidx2 = lax.broadcasted_iota(jnp.int32, (N, R), 0)

    for s in range(2):
        c = (p1f_ref if s == 0 else p2f_ref)[0]
        ccols = lax.dot_general(c, eye3, (((0,), (0,)), ((), ())),
                                preferred_element_type=jnp.float32,
                                precision=lax.Precision.HIGHEST)
        cx, cy, cz = ccols[:, 0:1], ccols[:, 1:2], ccols[:, 2:3]
        cn = (cx * cx + cy * cy) + cz * cz
        dot0 = lax.dot_general(c, q, (((0,), (0,)), ((), ())),
                               preferred_element_type=jnp.float32)
        d2_ref[...] = (qn + cn) - 2.0 * dot0
        off = (2 * b + s) * N
        poolv_ref[...] = jnp.full((POOL, R), BIG, jnp.float32)
        done_ref[0] = 0

        for rd in range(KNN):
            @pl.when(done_ref[0] == 0)
            def _round(rd=rd):
                d3 = d2_ref[...].reshape(CH, cs, R)
                cmin = jnp.min(d3, axis=1)

                def _extract():
                    wi = jnp.where(d3 == cmin[:, None, :], li3, cs)
                    aidx = jnp.min(wi, axis=1)
                    d2_ref[...] = jnp.where(li3 == aidx[:, None, :], BIG,
                                            d3).reshape(N, R)
                    poolv_ref[rd * CH:(rd + 1) * CH, :] = cmin
                    pooli_ref[rd * CH:(rd + 1) * CH, :] = chunk_iota * cs + aidx

                if rd == 0:
                    _extract()
                else:
                    gm = jnp.min(cmin, axis=0, keepdims=True)
                    ridx = jnp.min(jnp.where(d2_ref[...] == gm, gidx2,
                                             jnp.int32(1 << 30)),
                                   axis=0, keepdims=True)
                    cnt = jnp.zeros((1, R), jnp.int32)
                    for j2 in range(rd):
                        bv = poolv_ref[j2 * CH:(j2 + 1) * CH, :]
                        bi = pooli_ref[j2 * CH:(j2 + 1) * CH, :]
                        ahead = (bv < gm) | ((bv == gm) & (bi < ridx))
                        cnt = cnt + jnp.sum(ahead.astype(jnp.int32),
                                            axis=0, keepdims=True)
                    ok = jnp.min(cnt) >= KNN
                    done_ref[0] = ok.astype(jnp.int32)
                    pl.when(jnp.logical_not(ok))(_extract)

        pi = pooli_ref[...]
        for j in range(KNN):
            pv = poolv_ref[...]
            m = jnp.min(pv, axis=0, keepdims=True)
            idx = jnp.min(jnp.where(pv == m, pi, jnp.int32(1 << 30)),
                          axis=0, keepdims=True)
            poolv_ref[...] = jnp.where(pi == idx, BIG, pv)
            oidx_ref[0, s * KNN + j, :] = idx[0] + off


def _gather_body(idx_hbm, tx_hbm, ty_hbm, tz_hbm, gx_hbm, gy_hbm, gz_hbm,
                 idx_v, out_v, sem):
    wid = lax.axis_index("s") * _NC + lax.axis_index("c")
    base = wid * _PER_W
    pltpu.sync_copy(idx_hbm.at[pl.ds(base, _PER_W)], idx_v)
    grp = 16
    for tbl, dst in ((tx_hbm, gx_hbm), (ty_hbm, gy_hbm), (tz_hbm, gz_hbm)):
        def body(g, carry, tbl=tbl):
            gbase = pl.multiple_of(g * (grp * _CHUNK), 1024)
            copies = []
            for i in range(grp):
                o = gbase + i * _CHUNK
                cp = pltpu.make_async_copy(
                    tbl.at[idx_v.at[pl.ds(o, _CHUNK)]],
                    out_v.at[pl.ds(o, _CHUNK)],
                    sem)
                cp.start()
                copies.append(cp)
            for cp in copies:
                cp.wait()
            return carry
        lax.fori_loop(0, _NCHUNK // grp, body, 0)
        pltpu.sync_copy(out_v, dst.at[pl.ds(base, _PER_W)])


def _sc_gather(idx_flat, tx, ty, tz):
    mesh = plsc.VectorSubcoreMesh(core_axis_name="c", subcore_axis_name="s")
    fn = functools.partial(
        pl.kernel,
        mesh=mesh,
        out_type=[jax.ShapeDtypeStruct((_TOTAL_IDX,), jnp.float32)] * 3,
        scratch_types=[
            pltpu.VMEM((_PER_W,), jnp.int32),
            pltpu.VMEM((_PER_W,), jnp.float32),
            pltpu.SemaphoreType.DMA,
        ],
    )(_gather_body)
    return fn(idx_flat, tx, ty, tz)


def _mlp_kernel(q_ref, gx_ref, gy_ref, gz_ref,
                w0_ref, bn0_ref, w1_ref, bn1_ref, w2_ref, bn2_ref,
                out_ref, x0_ref):
    q = q_ref[0]
    gx = gx_ref[0]
    gy = gy_ref[0]
    gz = gz_ref[0]
    rx = gx - q[0:1, :]
    ry = gy - q[1:2, :]
    rz = gz - q[2:3, :]
    dist = jnp.sqrt((rx * rx + ry * ry + rz * rz) + 1e-12)
    for slot in range(NSLOT):
        sl = slice(slot * R, (slot + 1) * R)
        x0_ref[0:1, sl] = rx[slot:slot + 1, :]
        x0_ref[1:2, sl] = ry[slot:slot + 1, :]
        x0_ref[2:3, sl] = rz[slot:slot + 1, :]
        x0_ref[3:4, sl] = dist[slot:slot + 1, :]

    h = x0_ref[...]
    for w_ref, bn_ref in ((w0_ref, bn0_ref), (w1_ref, bn1_ref), (w2_ref, bn2_ref)):
        t = jnp.dot(w_ref[...], h, preferred_element_type=jnp.float32)
        t = t + bn_ref[:, 0:1]
        t = (t - bn_ref[:, 1:2]) / bn_ref[:, 2:3]
        t = t * bn_ref[:, 3:4] + bn_ref[:, 4:5]
        h = jnp.maximum(t, 0.0)
    sc = jnp.max(h, axis=0, keepdims=True)
    s32 = jnp.concatenate([sc[:, j * R:(j + 1) * R] for j in range(NSLOT)],
                          axis=0)
    mx = jnp.max(s32, axis=0, keepdims=True)
    e = jnp.exp(s32 - mx)
    w = e / jnp.sum(e, axis=0, keepdims=True)
    ox = jnp.sum(w * gx, axis=0, keepdims=True)
    oy = jnp.sum(w * gy, axis=0, keepdims=True)
    oz = jnp.sum(w * gz, axis=0, keepdims=True)
    out_ref[0] = jnp.concatenate([ox, oy, oz], axis=0)


def kernel(points1, points2, k, t,
           W0, b0, g0, be0, rm0, rv0,
           W1, b1, g1, be1, rm1, rv1,
           W2, b2, g2, be2, rm2, rv2):
    eps = 1e-3
    bn0 = jnp.stack([b0, rm0, jnp.sqrt(rv0 + eps), g0, be0], axis=1)
    bn1 = jnp.stack([b1, rm1, jnp.sqrt(rv1 + eps), g1, be1], axis=1)
    bn2 = jnp.stack([b2, rm2, jnp.sqrt(rv2 + eps), g2, be2], axis=1)

    B = points1.shape[0]
    oidx = pl.pallas_call(
        _topk_kernel,
        grid=(B, NB),
        in_specs=[
            pl.BlockSpec((1, 3, N), lambda b, i: (b, 0, 0)),
            pl.BlockSpec((1, 3, N), lambda b, i: (b, 0, 0)),
            pl.BlockSpec((1, 3, R), lambda b, i: (b, 0, i)),
        ],
        out_specs=pl.BlockSpec((1, NSLOT, R), lambda b, i: (b, 0, i)),
        out_shape=jax.ShapeDtypeStruct((B, NSLOT, N), jnp.int32),
        scratch_shapes=[
            pltpu.VMEM((N, R), jnp.float32),
            pltpu.VMEM((POOL, R), jnp.float32),
            pltpu.VMEM((POOL, R), jnp.int32),
            pltpu.SMEM((1,), jnp.int32),
        ],
    )(points1, points2, points1)

    tabs = jnp.stack([points1, points2], axis=1)
    tx = tabs[:, :, 0, :].reshape(-1)
    ty = tabs[:, :, 1, :].reshape(-1)
    tz = tabs[:, :, 2, :].reshape(-1)
    gx, gy, gz = _sc_gather(oidx.reshape(-1), tx, ty, tz)
    gx = gx.reshape(B, NSLOT, N)
    gy = gy.reshape(B, NSLOT, N)
    gz = gz.reshape(B, NSLOT, N)

    fused = pl.pallas_call(
        _mlp_kernel,
        grid=(B, NB),
        in_specs=[
            pl.BlockSpec((1, 3, R), lambda b, i: (b, 0, i)),
            pl.BlockSpec((1, NSLOT, R), lambda b, i: (b, 0, i)),
            pl.BlockSpec((1, NSLOT, R), lambda b, i: (b, 0, i)),
            pl.BlockSpec((1, NSLOT, R), lambda b, i: (b, 0, i)),
            pl.BlockSpec((64, 4), lambda b, i: (0, 0)),
            pl.BlockSpec((64, 5), lambda b, i: (0, 0)),
            pl.BlockSpec((64, 64), lambda b, i: (0, 0)),
            pl.BlockSpec((64, 5), lambda b, i: (0, 0)),
            pl.BlockSpec((128, 64), lambda b, i: (0, 0)),
            pl.BlockSpec((128, 5), lambda b, i: (0, 0)),
        ],
        out_specs=pl.BlockSpec((1, 3, R), lambda b, i: (b, 0, i)),
        out_shape=jax.ShapeDtypeStruct((B, 3, N), jnp.float32),
        scratch_shapes=[pltpu.VMEM((4, NSLOT * R), jnp.float32)],
    )(points1, gx, gy, gz, W0, bn0, W1, bn1, W2, bn2)
    return fused


# speedup vs baseline: 1.3165x; 1.3165x over previous
import functools

import jax
import jax.numpy as jnp
from jax import lax
from jax.experimental import pallas as pl
from jax.experimental.pallas import tpu as pltpu
from jax.experimental.pallas import tpu_sc as plsc

N = 4096
R = 256
NB = N // R
KNN = 16
NSLOT = 2 * KNN
BIG = 3.0e38

_NC, _NS = 2, 16
_NW = _NC * _NS
_TOTAL_IDX = 4 * NSLOT * N
_PER_W = _TOTAL_IDX // _NW
_CHUNK = 128
_NCHUNK = _PER_W // _CHUNK


def _topk_kernel(p1f_ref, p2f_ref, q_ref, oidx_ref, d2_ref):
    b = pl.program_id(0)
    q = q_ref[0]
    qn = jnp.sum(q * q, axis=0, keepdims=True)
    iota = lax.broadcasted_iota(jnp.int32, (N, R), 0)
    eye3 = jnp.eye(3, dtype=jnp.float32)

    for s in range(2):
        c = (p1f_ref if s == 0 else p2f_ref)[0]
        ccols = lax.dot_general(c, eye3, (((0,), (0,)), ((), ())),
                                preferred_element_type=jnp.float32,
                                precision=lax.Precision.HIGHEST)
        cx, cy, cz = ccols[:, 0:1], ccols[:, 1:2], ccols[:, 2:3]
        cn = (cx * cx + cy * cy) + cz * cz
        dot0 = lax.dot_general(c, q, (((0,), (0,)), ((), ())),
                               preferred_element_type=jnp.float32)
        d2_ref[...] = (qn + cn) - 2.0 * dot0
        off = (2 * b + s) * N
        for j in range(KNN):
            d2 = d2_ref[...]
            m = jnp.min(d2, axis=0, keepdims=True)
            idx = jnp.min(jnp.where(d2 == m, iota, N), axis=0, keepdims=True)
            d2_ref[...] = jnp.where(iota == idx, BIG, d2)
            oidx_ref[0, s * KNN + j, :] = idx[0] + off


def _gather_body(idx_hbm, tx_hbm, ty_hbm, tz_hbm, gx_hbm, gy_hbm, gz_hbm,
                 idx_v, out_v, sem):
    wid = lax.axis_index("s") * _NC + lax.axis_index("c")
    base = wid * _PER_W
    pltpu.sync_copy(idx_hbm.at[pl.ds(base, _PER_W)], idx_v)
    grp = 16
    for tbl, dst in ((tx_hbm, gx_hbm), (ty_hbm, gy_hbm), (tz_hbm, gz_hbm)):
        def body(g, carry, tbl=tbl):
            gbase = pl.multiple_of(g * (grp * _CHUNK), 1024)
            copies = []
            for i in range(grp):
                o = gbase + i * _CHUNK
                cp = pltpu.make_async_copy(
                    tbl.at[idx_v.at[pl.ds(o, _CHUNK)]],
                    out_v.at[pl.ds(o, _CHUNK)],
                    sem)
                cp.start()
                copies.append(cp)
            for cp in copies:
                cp.wait()
            return carry
        lax.fori_loop(0, _NCHUNK // grp, body, 0)
        pltpu.sync_copy(out_v, dst.at[pl.ds(base, _PER_W)])


def _sc_gather(idx_flat, tx, ty, tz):
    mesh = plsc.VectorSubcoreMesh(core_axis_name="c", subcore_axis_name="s")
    fn = functools.partial(
        pl.kernel,
        mesh=mesh,
        out_type=[jax.ShapeDtypeStruct((_TOTAL_IDX,), jnp.float32)] * 3,
        scratch_types=[
            pltpu.VMEM((_PER_W,), jnp.int32),
            pltpu.VMEM((_PER_W,), jnp.float32),
            pltpu.SemaphoreType.DMA,
        ],
    )(_gather_body)
    return fn(idx_flat, tx, ty, tz)


def _mlp_kernel(q_ref, gx_ref, gy_ref, gz_ref,
                w0_ref, bn0_ref, w1_ref, bn1_ref, w2_ref, bn2_ref,
                out_ref, x0_ref):
    q = q_ref[0]
    gx = gx_ref[0]
    gy = gy_ref[0]
    gz = gz_ref[0]
    rx = gx - q[0:1, :]
    ry = gy - q[1:2, :]
    rz = gz - q[2:3, :]
    dist = jnp.sqrt((rx * rx + ry * ry + rz * rz) + 1e-12)
    for slot in range(NSLOT):
        sl = slice(slot * R, (slot + 1) * R)
        x0_ref[0:1, sl] = rx[slot:slot + 1, :]
        x0_ref[1:2, sl] = ry[slot:slot + 1, :]
        x0_ref[2:3, sl] = rz[slot:slot + 1, :]
        x0_ref[3:4, sl] = dist[slot:slot + 1, :]

    h = x0_ref[...]
    for w_ref, bn_ref in ((w0_ref, bn0_ref), (w1_ref, bn1_ref), (w2_ref, bn2_ref)):
        t = jnp.dot(w_ref[...], h, preferred_element_type=jnp.float32)
        t = t + bn_ref[:, 0:1]
        t = (t - bn_ref[:, 1:2]) / bn_ref[:, 2:3]
        t = t * bn_ref[:, 3:4] + bn_ref[:, 4:5]
        h = jnp.maximum(t, 0.0)
    sc = jnp.max(h, axis=0, keepdims=True)
    s32 = jnp.concatenate([sc[:, j * R:(j + 1) * R] for j in range(NSLOT)],
                          axis=0)
    mx = jnp.max(s32, axis=0, keepdims=True)
    e = jnp.exp(s32 - mx)
    w = e / jnp.sum(e, axis=0, keepdims=True)
    ox = jnp.sum(w * gx, axis=0, keepdims=True)
    oy = jnp.sum(w * gy, axis=0, keepdims=True)
    oz = jnp.sum(w * gz, axis=0, keepdims=True)
    out_ref[0] = jnp.concatenate([ox, oy, oz], axis=0)


def kernel(points1, points2, k, t,
           W0, b0, g0, be0, rm0, rv0,
           W1, b1, g1, be1, rm1, rv1,
           W2, b2, g2, be2, rm2, rv2):
    eps = 1e-3
    bn0 = jnp.stack([b0, rm0, jnp.sqrt(rv0 + eps), g0, be0], axis=1)
    bn1 = jnp.stack([b1, rm1, jnp.sqrt(rv1 + eps), g1, be1], axis=1)
    bn2 = jnp.stack([b2, rm2, jnp.sqrt(rv2 + eps), g2, be2], axis=1)

    B = points1.shape[0]
    oidx = pl.pallas_call(
        _topk_kernel,
        grid=(B, NB),
        in_specs=[
            pl.BlockSpec((1, 3, N), lambda b, i: (b, 0, 0)),
            pl.BlockSpec((1, 3, N), lambda b, i: (b, 0, 0)),
            pl.BlockSpec((1, 3, R), lambda b, i: (b, 0, i)),
        ],
        out_specs=pl.BlockSpec((1, NSLOT, R), lambda b, i: (b, 0, i)),
        out_shape=jax.ShapeDtypeStruct((B, NSLOT, N), jnp.int32),
        scratch_shapes=[pltpu.VMEM((N, R), jnp.float32)],
    )(points1, points2, points1)

    tabs = jnp.stack([points1, points2], axis=1)
    tx = tabs[:, :, 0, :].reshape(-1)
    ty = tabs[:, :, 1, :].reshape(-1)
    tz = tabs[:, :, 2, :].reshape(-1)
    gx, gy, gz = _sc_gather(oidx.reshape(-1), tx, ty, tz)
    gx = gx.reshape(B, NSLOT, N)
    gy = gy.reshape(B, NSLOT, N)
    gz = gz.reshape(B, NSLOT, N)

    fused = pl.pallas_call(
        _mlp_kernel,
        grid=(B, NB),
        in_specs=[
            pl.BlockSpec((1, 3, R), lambda b, i: (b, 0, i)),
            pl.BlockSpec((1, NSLOT, R), lambda b, i: (b, 0, i)),
            pl.BlockSpec((1, NSLOT, R), lambda b, i: (b, 0, i)),
            pl.BlockSpec((1, NSLOT, R), lambda b, i: (b, 0, i)),
            pl.BlockSpec((64, 4), lambda b, i: (0, 0)),
            pl.BlockSpec((64, 5), lambda b, i: (0, 0)),
            pl.BlockSpec((64, 64), lambda b, i: (0, 0)),
            pl.BlockSpec((64, 5), lambda b, i: (0, 0)),
            pl.BlockSpec((128, 64), lambda b, i: (0, 0)),
            pl.BlockSpec((128, 5), lambda b, i: (0, 0)),
        ],
        out_specs=pl.BlockSpec((1, 3, R), lambda b, i: (b, 0, i)),
        out_shape=jax.ShapeDtypeStruct((B, 3, N), jnp.float32),
        scratch_shapes=[pltpu.VMEM((4, NSLOT * R), jnp.float32)],
    )(points1, gx, gy, gz, W0, bn0, W1, bn1, W2, bn2)
    return fused


# speedup vs baseline: 1.5836x; 1.2029x over previous
import functools

import jax
import jax.numpy as jnp
from jax import lax
from jax.experimental import pallas as pl
from jax.experimental.pallas import tpu as pltpu
from jax.experimental.pallas import tpu_sc as plsc

N = 4096
R = 512
NB = N // R
KNN = 16
NSLOT = 2 * KNN
BIG = 3.0e38

_NC, _NS = 2, 16
_NW = _NC * _NS
_TOTAL_IDX = 4 * NSLOT * N
_PER_W = _TOTAL_IDX // _NW
_CHUNK = 128
_NCHUNK = _PER_W // _CHUNK


def _topk_kernel(p1f_ref, p2f_ref, q_ref, oidx_ref, d2_ref):
    b = pl.program_id(0)
    q = q_ref[0]
    qn = jnp.sum(q * q, axis=0, keepdims=True)
    iota = lax.broadcasted_iota(jnp.int32, (N, R), 0)
    eye3 = jnp.eye(3, dtype=jnp.float32)

    for s in range(2):
        c = (p1f_ref if s == 0 else p2f_ref)[0]
        ccols = lax.dot_general(c, eye3, (((0,), (0,)), ((), ())),
                                preferred_element_type=jnp.float32,
                                precision=lax.Precision.HIGHEST)
        cx, cy, cz = ccols[:, 0:1], ccols[:, 1:2], ccols[:, 2:3]
        cn = (cx * cx + cy * cy) + cz * cz
        dot0 = lax.dot_general(c, q, (((0,), (0,)), ((), ())),
                               preferred_element_type=jnp.float32)
        d2_ref[...] = (qn + cn) - 2.0 * dot0
        off = (2 * b + s) * N
        for j in range(KNN):
            d2 = d2_ref[...]
            m = jnp.min(d2, axis=0, keepdims=True)
            idx = jnp.min(jnp.where(d2 == m, iota, N), axis=0, keepdims=True)
            d2_ref[...] = jnp.where(iota == idx, BIG, d2)
            oidx_ref[0, s * KNN + j, :] = idx[0] + off


def _gather_body(idx_hbm, tx_hbm, ty_hbm, tz_hbm, gx_hbm, gy_hbm, gz_hbm,
                 idx_v, out_v, sem):
    wid = lax.axis_index("s") * _NC + lax.axis_index("c")
    base = wid * _PER_W
    pltpu.sync_copy(idx_hbm.at[pl.ds(base, _PER_W)], idx_v)
    grp = 16
    for tbl, dst in ((tx_hbm, gx_hbm), (ty_hbm, gy_hbm), (tz_hbm, gz_hbm)):
        def body(g, carry, tbl=tbl):
            gbase = pl.multiple_of(g * (grp * _CHUNK), 1024)
            copies = []
            for i in range(grp):
                o = gbase + i * _CHUNK
                cp = pltpu.make_async_copy(
                    tbl.at[idx_v.at[pl.ds(o, _CHUNK)]],
                    out_v.at[pl.ds(o, _CHUNK)],
                    sem)
                cp.start()
                copies.append(cp)
            for cp in copies:
                cp.wait()
            return carry
        lax.fori_loop(0, _NCHUNK // grp, body, 0)
        pltpu.sync_copy(out_v, dst.at[pl.ds(base, _PER_W)])


def _sc_gather(idx_flat, tx, ty, tz):
    mesh = plsc.VectorSubcoreMesh(core_axis_name="c", subcore_axis_name="s")
    fn = functools.partial(
        pl.kernel,
        mesh=mesh,
        out_type=[jax.ShapeDtypeStruct((_TOTAL_IDX,), jnp.float32)] * 3,
        scratch_types=[
            pltpu.VMEM((_PER_W,), jnp.int32),
            pltpu.VMEM((_PER_W,), jnp.float32),
            pltpu.SemaphoreType.DMA,
        ],
    )(_gather_body)
    return fn(idx_flat, tx, ty, tz)


def _mlp_kernel(q_ref, gx_ref, gy_ref, gz_ref,
                w0_ref, bn0_ref, w1_ref, bn1_ref, w2_ref, bn2_ref,
                out_ref, x0_ref):
    q = q_ref[0]
    gx = gx_ref[0]
    gy = gy_ref[0]
    gz = gz_ref[0]
    rx = gx - q[0:1, :]
    ry = gy - q[1:2, :]
    rz = gz - q[2:3, :]
    dist = jnp.sqrt((rx * rx + ry * ry + rz * rz) + 1e-12)
    for slot in range(NSLOT):
        sl = slice(slot * R, (slot + 1) * R)
        x0_ref[0:1, sl] = rx[slot:slot + 1, :]
        x0_ref[1:2, sl] = ry[slot:slot + 1, :]
        x0_ref[2:3, sl] = rz[slot:slot + 1, :]
        x0_ref[3:4, sl] = dist[slot:slot + 1, :]

    h = x0_ref[...]
    for w_ref, bn_ref in ((w0_ref, bn0_ref), (w1_ref, bn1_ref), (w2_ref, bn2_ref)):
        t = jnp.dot(w_ref[...], h, preferred_element_type=jnp.float32)
        t = t + bn_ref[:, 0:1]
        t = (t - bn_ref[:, 1:2]) / bn_ref[:, 2:3]
        t = t * bn_ref[:, 3:4] + bn_ref[:, 4:5]
        h = jnp.maximum(t, 0.0)
    sc = jnp.max(h, axis=0, keepdims=True)
    s32 = jnp.concatenate([sc[:, j * R:(j + 1) * R] for j in range(NSLOT)],
                          axis=0)
    mx = jnp.max(s32, axis=0, keepdims=True)
    e = jnp.exp(s32 - mx)
    w = e / jnp.sum(e, axis=0, keepdims=True)
    ox = jnp.sum(w * gx, axis=0, keepdims=True)
    oy = jnp.sum(w * gy, axis=0, keepdims=True)
    oz = jnp.sum(w * gz, axis=0, keepdims=True)
    out_ref[0] = jnp.concatenate([ox, oy, oz], axis=0)


def kernel(points1, points2, k, t,
           W0, b0, g0, be0, rm0, rv0,
           W1, b1, g1, be1, rm1, rv1,
           W2, b2, g2, be2, rm2, rv2):
    eps = 1e-3
    bn0 = jnp.stack([b0, rm0, jnp.sqrt(rv0 + eps), g0, be0], axis=1)
    bn1 = jnp.stack([b1, rm1, jnp.sqrt(rv1 + eps), g1, be1], axis=1)
    bn2 = jnp.stack([b2, rm2, jnp.sqrt(rv2 + eps), g2, be2], axis=1)

    B = points1.shape[0]
    oidx = pl.pallas_call(
        _topk_kernel,
        grid=(B, NB),
        in_specs=[
            pl.BlockSpec((1, 3, N), lambda b, i: (b, 0, 0)),
            pl.BlockSpec((1, 3, N), lambda b, i: (b, 0, 0)),
            pl.BlockSpec((1, 3, R), lambda b, i: (b, 0, i)),
        ],
        out_specs=pl.BlockSpec((1, NSLOT, R), lambda b, i: (b, 0, i)),
        out_shape=jax.ShapeDtypeStruct((B, NSLOT, N), jnp.int32),
        scratch_shapes=[pltpu.VMEM((N, R), jnp.float32)],
    )(points1, points2, points1)

    tabs = jnp.stack([points1, points2], axis=1)
    tx = tabs[:, :, 0, :].reshape(-1)
    ty = tabs[:, :, 1, :].reshape(-1)
    tz = tabs[:, :, 2, :].reshape(-1)
    gx, gy, gz = _sc_gather(oidx.reshape(-1), tx, ty, tz)
    gx = gx.reshape(B, NSLOT, N)
    gy = gy.reshape(B, NSLOT, N)
    gz = gz.reshape(B, NSLOT, N)

    fused = pl.pallas_call(
        _mlp_kernel,
        grid=(B, NB),
        in_specs=[
            pl.BlockSpec((1, 3, R), lambda b, i: (b, 0, i)),
            pl.BlockSpec((1, NSLOT, R), lambda b, i: (b, 0, i)),
            pl.BlockSpec((1, NSLOT, R), lambda b, i: (b, 0, i)),
            pl.BlockSpec((1, NSLOT, R), lambda b, i: (b, 0, i)),
            pl.BlockSpec((64, 4), lambda b, i: (0, 0)),
            pl.BlockSpec((64, 5), lambda b, i: (0, 0)),
            pl.BlockSpec((64, 64), lambda b, i: (0, 0)),
            pl.BlockSpec((64, 5), lambda b, i: (0, 0)),
            pl.BlockSpec((128, 64), lambda b, i: (0, 0)),
            pl.BlockSpec((128, 5), lambda b, i: (0, 0)),
        ],
        out_specs=pl.BlockSpec((1, 3, R), lambda b, i: (b, 0, i)),
        out_shape=jax.ShapeDtypeStruct((B, 3, N), jnp.float32),
        scratch_shapes=[pltpu.VMEM((4, NSLOT * R), jnp.float32)],
    )(points1, gx, gy, gz, W0, bn0, W1, bn1, W2, bn2)
    return fused
